# baseline (speedup 1.0000x reference)
import functools

import jax
import jax.numpy as jnp
from jax import lax
from jax.experimental import pallas as pl
from jax.experimental.pallas import tpu as pltpu
from jax.experimental.pallas import tpu_sc as plsc

B = 16384
N = 1000000
E = 32
NC = 2
NS = 16
L = 16
NW = NC * NS
BPW = B // NW
CHUNK = 128
NCHUNK = BPW // CHUNK
NGROUP = BPW // L


def _sc_body(uidx_hbm, bidx_hbm, utab_hbm, btab_hbm, ubias_hbm, bbias_hbm,
             out_hbm, uidx_v, bidx_v, uflat_v, bflat_v, urowsT_v, browsT_v,
             ub_v, bb_v, out_v, sem):
    wid = lax.axis_index("s") * NC + lax.axis_index("c")
    base = wid * BPW

    pltpu.sync_copy(uidx_hbm.at[pl.ds(base, BPW)], uidx_v)
    pltpu.sync_copy(bidx_hbm.at[pl.ds(base, BPW)], bidx_v)

    def shift(s, carry):
        v = uidx_v[pl.ds(s * L, L)]
        w = bidx_v[pl.ds(s * L, L)]
        for e in range(E):
            uflat_v[e, pl.ds(s * L, L)] = v + e * N
            bflat_v[e, pl.ds(s * L, L)] = w + e * N
        return carry

    lax.fori_loop(0, BPW // L, shift, 0)

    copies = []
    for c in range(NCHUNK):
        sl = pl.ds(c * CHUNK, CHUNK)
        copies.append(pltpu.async_copy(ubias_hbm.at[uidx_v.at[sl]],
                                       ub_v.at[sl], sem))
        copies.append(pltpu.async_copy(bbias_hbm.at[bidx_v.at[sl]],
                                       bb_v.at[sl], sem))
        for e in range(E):
            copies.append(pltpu.async_copy(
                utab_hbm.at[uflat_v.at[e].at[sl]],
                urowsT_v.at[e].at[sl], sem))
            copies.append(pltpu.async_copy(
                btab_hbm.at[bflat_v.at[e].at[sl]],
                browsT_v.at[e].at[sl], sem))
    for cp in copies:
        cp.wait()

    def group(g, carry):
        pbase = g * L
        sl = pl.ds(pbase, L)
        acc = urowsT_v[0, sl] * browsT_v[0, sl]
        for e in range(1, E):
            acc = acc + urowsT_v[e, sl] * browsT_v[e, sl]
        x = acc + ub_v[sl] + bb_v[sl]
        out_v[sl] = 1.0 / (1.0 + jnp.exp(-x))
        return carry

    lax.fori_loop(0, NGROUP, group, 0)

    pltpu.sync_copy(out_v, out_hbm.at[pl.ds(base, BPW)])


_sc_call = pl.kernel(
    _sc_body,
    out_type=jax.ShapeDtypeStruct((B,), jnp.float32),
    mesh=plsc.VectorSubcoreMesh(core_axis_name="c", subcore_axis_name="s"),
    compiler_params=pltpu.CompilerParams(needs_layout_passes=False,
                                         use_tc_tiling_on_sc=False),
    scratch_types=[
        pltpu.VMEM((BPW,), jnp.int32),
        pltpu.VMEM((BPW,), jnp.int32),
        pltpu.VMEM((E, BPW), jnp.int32),
        pltpu.VMEM((E, BPW), jnp.int32),
        pltpu.VMEM((E, BPW), jnp.float32),
        pltpu.VMEM((E, BPW), jnp.float32),
        pltpu.VMEM((BPW,), jnp.float32),
        pltpu.VMEM((BPW,), jnp.float32),
        pltpu.VMEM((BPW,), jnp.float32),
        pltpu.SemaphoreType.DMA,
    ],
)


def kernel(inputs, user_embedding, user_bias, book_embedding, book_bias):
    uidx = inputs[:, 0]
    bidx = inputs[:, 1]
    uflat = user_embedding.T.reshape(-1)
    bflat = book_embedding.T.reshape(-1)
    out = _sc_call(uidx, bidx, uflat, bflat,
                   user_bias[:, 0], book_bias[:, 0])
    return out.reshape(B, 1)
